# Initial kernel scaffold; baseline (speedup 1.0000x reference)
import jax
import jax.numpy as jnp
from jax.experimental import pallas as pl


def kernel(x, edge_index, weight, bias):
    raise NotImplementedError("write your pallas kernel here")


# speedup vs baseline: 3.6183x; 3.6183x over previous
import functools

import jax
import jax.numpy as jnp
from jax import lax
from jax.experimental import pallas as pl
from jax.experimental.pallas import tpu as pltpu
from jax.experimental.pallas import tpu_sc as plsc

MIN_NORM = 1e-15
BALL_EPS = 4e-3
_MAXNORM = 1.0 - BALL_EPS

BLK = 1000
NC = 2
NS = 16
K = 80
DEGW = 16


def _rownorm(x):
    return jnp.maximum(jnp.sqrt(jnp.sum(x * x, axis=-1, keepdims=True)), MIN_NORM)


def _artanh(x):
    x = jnp.clip(x, -1.0 + 1e-7, 1.0 - 1e-7)
    return 0.5 * jnp.log((1.0 + x) / (1.0 - x))


def _proj(x):
    n = _rownorm(x)
    return jnp.where(n > _MAXNORM, x / n * _MAXNORM, x)


def _expmap0(u):
    n = _rownorm(u)
    return jnp.tanh(n) * u / n


def _logmap0(p):
    n = _rownorm(p)
    return _artanh(n) / n * p


def _pre_body(x_ref, w_ref, b_ref, o_ref):
    x = x_ref[...]
    w = w_ref[...]
    b = b_ref[...]
    x_norm = _rownorm(x)
    mx = lax.dot_general(x, w, (((1,), (1,)), ((), ())),
                         preferred_element_type=jnp.float32)
    mx_norm = _rownorm(mx)
    res = jnp.tanh(mx_norm / x_norm * _artanh(x_norm)) * mx / mx_norm
    res = jnp.where(jnp.all(mx == 0.0, axis=-1, keepdims=True), 0.0, res)
    res = _proj(res)
    hb = _proj(_expmap0(b))
    x2 = jnp.sum(res * res, axis=-1, keepdims=True)
    y2 = jnp.sum(hb * hb, axis=-1, keepdims=True)
    xy = jnp.sum(res * hb, axis=-1, keepdims=True)
    num = (1.0 + 2.0 * xy + y2) * res + (1.0 - x2) * hb
    den = 1.0 + 2.0 * xy + x2 * y2
    res = _proj(num / jnp.maximum(den, MIN_NORM))
    o_ref[...] = _logmap0(res)


def _post_body(sup_ref, deg_ref, o_ref):
    s = sup_ref[...]
    deg = deg_ref[:, 0:1]
    support_t = s * (1.0 / jnp.maximum(deg, 1.0))
    h = _proj(_expmap0(support_t))
    xt = jnp.maximum(_logmap0(h), 0.0)
    o_ref[...] = _proj(_expmap0(xt))


def _pre(x, w, b2):
    n, d = x.shape
    return pl.pallas_call(
        _pre_body,
        grid=(n // BLK,),
        in_specs=[pl.BlockSpec((BLK, d), lambda i: (i, 0)),
                  pl.BlockSpec((d, d), lambda i: (0, 0)),
                  pl.BlockSpec((1, d), lambda i: (0, 0))],
        out_specs=pl.BlockSpec((BLK, d), lambda i: (i, 0)),
        out_shape=jax.ShapeDtypeStruct((n, d), jnp.float32),
    )(x, w, b2)


def _post(sup, degp):
    n, d = sup.shape
    return pl.pallas_call(
        _post_body,
        grid=(n // BLK,),
        in_specs=[pl.BlockSpec((BLK, d), lambda i: (i, 0)),
                  pl.BlockSpec((BLK, d), lambda i: (i, 0))],
        out_specs=pl.BlockSpec((BLK, d), lambda i: (i, 0)),
        out_shape=jax.ShapeDtypeStruct((n, d), jnp.float32),
    )(sup, degp)


def _localize_dst(dstb, nchunks, cid, half, garb):
    lo = cid * half

    @pl.loop(0, nchunks)
    def _(c2):
        @pl.loop(0, K // 16)
        def _(j):
            v = dstb[c2, pl.ds(j * 16, 16)] - lo
            m = (v >= 0) & (v < half)
            dstb[c2, pl.ds(j * 16, 16)] = jnp.where(m, v, garb)


def _deg(ei4, n, d):
    nchunks = ei4.shape[2]
    half = n // NC
    nacc = half + 8
    garb = half
    base = 8 * (half // (8 * NS))
    remz = nacc - base * NS
    remo = half - base * NS
    zchunk = 104
    mesh = plsc.VectorSubcoreMesh(core_axis_name="c", subcore_axis_name="s")

    @functools.partial(
        pl.kernel,
        out_type=jax.ShapeDtypeStruct((n, d), jnp.float32),
        mesh=mesh,
        scratch_types=[
            pltpu.VMEM((nchunks, K), jnp.int32),
            pltpu.VMEM((K, d), jnp.float32),
            pltpu.VMEM((zchunk, d), jnp.float32),
            pltpu.VMEM_SHARED((nacc, d), jnp.float32),
        ],
    )
    def k(ei_hbm, deg_hbm, dstb, ones, zbd, accd):
        cid = lax.axis_index("c")
        sid = lax.axis_index("s")
        zv = jnp.zeros((16,), jnp.float32)

        @pl.loop(0, zchunk)
        def _(i):
            @pl.loop(0, d // 16)
            def _(j):
                zbd[i, pl.ds(j * 16, 16)] = zv

        @pl.loop(0, K)
        def _(i):
            @pl.loop(0, d // 16)
            def _(j):
                ones[i, pl.ds(j * 16, 16)] = zv + 1.0

        rbase = sid * base

        @pl.loop(0, base // zchunk)
        def _(i):
            pltpu.sync_copy(zbd, accd.at[pl.ds(rbase + i * zchunk, zchunk)])

        @pl.when(sid == NS - 1)
        def _():
            pltpu.sync_copy(zbd.at[pl.ds(0, remz)],
                            accd.at[pl.ds(nacc - remz, remz)])

        pltpu.sync_copy(ei_hbm.at[1, sid], dstb)
        _localize_dst(dstb, nchunks, cid, half, garb)
        plsc.subcore_barrier()

        @pl.loop(0, nchunks)
        def _(c2):
            pltpu.sync_copy(ones, accd.at[dstb.at[c2]], add=True)

        plsc.subcore_barrier()

        obase = cid * half

        @pl.loop(0, base // zchunk)
        def _(i):
            r0 = rbase + i * zchunk
            pltpu.sync_copy(accd.at[pl.ds(r0, zchunk)], zbd)
            pltpu.sync_copy(zbd, deg_hbm.at[pl.ds(obase + r0, zchunk)])

        @pl.when(sid == NS - 1)
        def _():
            r0 = base * NS
            pltpu.sync_copy(accd.at[pl.ds(r0, remo)], zbd.at[pl.ds(0, remo)])
            pltpu.sync_copy(zbd.at[pl.ds(0, remo)],
                            deg_hbm.at[pl.ds(obase + r0, remo)])

    return k(ei4)


def _sup(ei4, xt):
    n, d = xt.shape
    nchunks = ei4.shape[2]
    half = n // NC
    nacc = half + 8
    garb = half
    base = 8 * (half // (8 * NS))
    remz = nacc - base * NS
    remo = half - base * NS
    zchunk = 104
    mesh = plsc.VectorSubcoreMesh(core_axis_name="c", subcore_axis_name="s")

    @functools.partial(
        pl.kernel,
        out_type=jax.ShapeDtypeStruct((n, d), jnp.float32),
        mesh=mesh,
        scratch_types=[
            pltpu.VMEM((nchunks, K), jnp.int32),
            pltpu.VMEM((nchunks, K), jnp.int32),
            pltpu.VMEM((K, d), jnp.float32),
            pltpu.VMEM((zchunk, d), jnp.float32),
            pltpu.VMEM_SHARED((nacc, d), jnp.float32),
        ],
    )
    def k(ei_hbm, xt_hbm, sup_hbm, srcb, dstb, rows, zb, acc):
        cid = lax.axis_index("c")
        sid = lax.axis_index("s")
        zv = jnp.zeros((16,), jnp.float32)

        @pl.loop(0, zchunk)
        def _(i):
            @pl.loop(0, d // 16)
            def _(j):
                zb[i, pl.ds(j * 16, 16)] = zv

        rbase = sid * base

        @pl.loop(0, base // zchunk)
        def _(i):
            pltpu.sync_copy(zb, acc.at[pl.ds(rbase + i * zchunk, zchunk)])

        @pl.when(sid == NS - 1)
        def _():
            pltpu.sync_copy(zb.at[pl.ds(0, remz)],
                            acc.at[pl.ds(nacc - remz, remz)])

        pltpu.sync_copy(ei_hbm.at[0, sid], srcb)
        pltpu.sync_copy(ei_hbm.at[1, sid], dstb)
        _localize_dst(dstb, nchunks, cid, half, garb)
        plsc.subcore_barrier()

        @pl.loop(0, nchunks)
        def _(c2):
            pltpu.sync_copy(xt_hbm.at[srcb.at[c2]], rows)
            pltpu.sync_copy(rows, acc.at[dstb.at[c2]], add=True)

        plsc.subcore_barrier()

        obase = cid * half

        @pl.loop(0, base // zchunk)
        def _(i):
            r0 = rbase + i * zchunk
            pltpu.sync_copy(acc.at[pl.ds(r0, zchunk)], zb)
            pltpu.sync_copy(zb, sup_hbm.at[pl.ds(obase + r0, zchunk)])

        @pl.when(sid == NS - 1)
        def _():
            r0 = base * NS
            pltpu.sync_copy(acc.at[pl.ds(r0, remo)], zb.at[pl.ds(0, remo)])
            pltpu.sync_copy(zb.at[pl.ds(0, remo)],
                            sup_hbm.at[pl.ds(obase + r0, remo)])

    return k(ei4, xt)


def kernel(x, edge_index, weight, bias):
    n, d = x.shape
    e = edge_index.shape[1]
    ei4 = edge_index.reshape(2, NS, e // (NS * K), K)
    degp = _deg(ei4, n, d)
    xt = _pre(x, weight, bias.reshape(1, d))
    sup = _sup(ei4, xt)
    return _post(sup, degp)


# speedup vs baseline: 4.2500x; 1.1746x over previous
import functools

import jax
import jax.numpy as jnp
from jax import lax
from jax.experimental import pallas as pl
from jax.experimental.pallas import tpu as pltpu
from jax.experimental.pallas import tpu_sc as plsc

MIN_NORM = 1e-15
BALL_EPS = 4e-3
_MAXNORM = 1.0 - BALL_EPS

BLK = 1000
NC = 2
NS = 16
K = 80
DEGW = 16
NOUT = 8
NBUF = 2


def _rownorm(x):
    return jnp.maximum(jnp.sqrt(jnp.sum(x * x, axis=-1, keepdims=True)), MIN_NORM)


def _artanh(x):
    x = jnp.clip(x, -1.0 + 1e-7, 1.0 - 1e-7)
    return 0.5 * jnp.log((1.0 + x) / (1.0 - x))


def _proj(x):
    n = _rownorm(x)
    return jnp.where(n > _MAXNORM, x / n * _MAXNORM, x)


def _expmap0(u):
    n = _rownorm(u)
    return jnp.tanh(n) * u / n


def _logmap0(p):
    n = _rownorm(p)
    return _artanh(n) / n * p


def _pre_body(x_ref, w_ref, b_ref, o_ref):
    x = x_ref[...]
    w = w_ref[...]
    b = b_ref[...]
    x_norm = _rownorm(x)
    mx = lax.dot_general(x, w, (((1,), (1,)), ((), ())),
                         preferred_element_type=jnp.float32)
    mx_norm = _rownorm(mx)
    res = jnp.tanh(mx_norm / x_norm * _artanh(x_norm)) * mx / mx_norm
    res = jnp.where(jnp.all(mx == 0.0, axis=-1, keepdims=True), 0.0, res)
    res = _proj(res)
    hb = _proj(_expmap0(b))
    x2 = jnp.sum(res * res, axis=-1, keepdims=True)
    y2 = jnp.sum(hb * hb, axis=-1, keepdims=True)
    xy = jnp.sum(res * hb, axis=-1, keepdims=True)
    num = (1.0 + 2.0 * xy + y2) * res + (1.0 - x2) * hb
    den = 1.0 + 2.0 * xy + x2 * y2
    res = _proj(num / jnp.maximum(den, MIN_NORM))
    o_ref[...] = _logmap0(res)


def _post_body(sup_ref, deg_ref, o_ref):
    s = sup_ref[...]
    deg = deg_ref[:, 0:1]
    support_t = s * (1.0 / jnp.maximum(deg, 1.0))
    h = _proj(_expmap0(support_t))
    xt = jnp.maximum(_logmap0(h), 0.0)
    o_ref[...] = _proj(_expmap0(xt))


def _pre(x, w, b2):
    n, d = x.shape
    return pl.pallas_call(
        _pre_body,
        grid=(n // BLK,),
        in_specs=[pl.BlockSpec((BLK, d), lambda i: (i, 0)),
                  pl.BlockSpec((d, d), lambda i: (0, 0)),
                  pl.BlockSpec((1, d), lambda i: (0, 0))],
        out_specs=pl.BlockSpec((BLK, d), lambda i: (i, 0)),
        out_shape=jax.ShapeDtypeStruct((n, d), jnp.float32),
    )(x, w, b2)


def _post(sup, degp):
    n, d = sup.shape
    return pl.pallas_call(
        _post_body,
        grid=(n // BLK,),
        in_specs=[pl.BlockSpec((BLK, d), lambda i: (i, 0)),
                  pl.BlockSpec((BLK, d), lambda i: (i, 0))],
        out_specs=pl.BlockSpec((BLK, d), lambda i: (i, 0)),
        out_shape=jax.ShapeDtypeStruct((n, d), jnp.float32),
    )(sup, degp)


def _localize_dst(dstb, nchunks, cid, half, garb):
    lo = cid * half

    @pl.loop(0, nchunks)
    def _(c2):
        @pl.loop(0, K // 16)
        def _(j):
            v = dstb[c2, pl.ds(j * 16, 16)] - lo
            m = (v >= 0) & (v < half)
            dstb[c2, pl.ds(j * 16, 16)] = jnp.where(m, v, garb)


def _deg(ei4, n, d):
    nchunks = ei4.shape[2]
    half = n // NC
    nacc = half + 8
    garb = half
    base = 8 * (half // (8 * NS))
    remz = nacc - base * NS
    remo = half - base * NS
    zchunk = 104
    mesh = plsc.VectorSubcoreMesh(core_axis_name="c", subcore_axis_name="s")

    @functools.partial(
        pl.kernel,
        out_type=jax.ShapeDtypeStruct((n, d), jnp.float32),
        mesh=mesh,
        scratch_types=[
            pltpu.VMEM((nchunks, K), jnp.int32),
            pltpu.VMEM((K, d), jnp.float32),
            pltpu.VMEM((zchunk, d), jnp.float32),
            pltpu.VMEM_SHARED((nacc, d), jnp.float32),
            pltpu.SemaphoreType.DMA((NOUT,)),
        ],
    )
    def k(ei_hbm, deg_hbm, dstb, ones, zbd, accd, ssem):
        cid = lax.axis_index("c")
        sid = lax.axis_index("s")
        zv = jnp.zeros((16,), jnp.float32)

        @pl.loop(0, zchunk)
        def _(i):
            @pl.loop(0, d // 16)
            def _(j):
                zbd[i, pl.ds(j * 16, 16)] = zv

        @pl.loop(0, K)
        def _(i):
            @pl.loop(0, d // 16)
            def _(j):
                ones[i, pl.ds(j * 16, 16)] = zv + 1.0

        rbase = sid * base

        @pl.loop(0, base // zchunk)
        def _(i):
            pltpu.sync_copy(zbd, accd.at[pl.ds(rbase + i * zchunk, zchunk)])

        @pl.when(sid == NS - 1)
        def _():
            pltpu.sync_copy(zbd.at[pl.ds(0, remz)],
                            accd.at[pl.ds(nacc - remz, remz)])

        pltpu.sync_copy(ei_hbm.at[1, sid], dstb)
        _localize_dst(dstb, nchunks, cid, half, garb)
        plsc.subcore_barrier()

        @pl.loop(0, nchunks)
        def _(c2):
            s = lax.rem(c2, NOUT)

            @pl.when(c2 >= NOUT)
            def _():
                pltpu.make_async_copy(
                    ones, accd.at[dstb.at[c2 - NOUT]], ssem.at[s]).wait()

            pltpu.async_copy(ones, accd.at[dstb.at[c2]], ssem.at[s],
                             add=True)

        @pl.loop(nchunks - NOUT, nchunks)
        def _(m):
            pltpu.make_async_copy(
                ones, accd.at[dstb.at[m]], ssem.at[lax.rem(m, NOUT)]).wait()

        plsc.subcore_barrier()

        obase = cid * half

        @pl.loop(0, base // zchunk)
        def _(i):
            r0 = rbase + i * zchunk
            pltpu.sync_copy(accd.at[pl.ds(r0, zchunk)], zbd)
            pltpu.sync_copy(zbd, deg_hbm.at[pl.ds(obase + r0, zchunk)])

        @pl.when(sid == NS - 1)
        def _():
            r0 = base * NS
            pltpu.sync_copy(accd.at[pl.ds(r0, remo)], zbd.at[pl.ds(0, remo)])
            pltpu.sync_copy(zbd.at[pl.ds(0, remo)],
                            deg_hbm.at[pl.ds(obase + r0, remo)])

    return k(ei4)


def _sup(ei4, xt):
    n, d = xt.shape
    nchunks = ei4.shape[2]
    half = n // NC
    nacc = half + 8
    garb = half
    base = 8 * (half // (8 * NS))
    remz = nacc - base * NS
    remo = half - base * NS
    zchunk = 24
    mesh = plsc.VectorSubcoreMesh(core_axis_name="c", subcore_axis_name="s")

    @functools.partial(
        pl.kernel,
        out_type=jax.ShapeDtypeStruct((n, d), jnp.float32),
        mesh=mesh,
        scratch_types=[
            pltpu.VMEM((nchunks, K), jnp.int32),
            pltpu.VMEM((nchunks, K), jnp.int32),
            pltpu.VMEM((NBUF, K, d), jnp.float32),
            pltpu.VMEM((zchunk, d), jnp.float32),
            pltpu.VMEM_SHARED((nacc, d), jnp.float32),
            pltpu.SemaphoreType.DMA((NBUF,)),
            pltpu.SemaphoreType.DMA((NBUF,)),
        ],
    )
    def k(ei_hbm, xt_hbm, sup_hbm, srcb, dstb, rows, zb, acc, gsem, ssem):
        cid = lax.axis_index("c")
        sid = lax.axis_index("s")
        zv = jnp.zeros((16,), jnp.float32)

        @pl.loop(0, zchunk)
        def _(i):
            @pl.loop(0, d // 16)
            def _(j):
                zb[i, pl.ds(j * 16, 16)] = zv

        rbase = sid * base

        @pl.loop(0, base // zchunk)
        def _(i):
            pltpu.sync_copy(zb, acc.at[pl.ds(rbase + i * zchunk, zchunk)])

        @pl.when(sid == NS - 1)
        def _():
            pltpu.sync_copy(zb.at[pl.ds(0, remz)],
                            acc.at[pl.ds(nacc - remz, remz)])

        pltpu.sync_copy(ei_hbm.at[0, sid], srcb)
        pltpu.sync_copy(ei_hbm.at[1, sid], dstb)
        _localize_dst(dstb, nchunks, cid, half, garb)
        plsc.subcore_barrier()

        hlf = NBUF // 2
        for b in range(NBUF):
            pltpu.async_copy(xt_hbm.at[srcb.at[b]], rows.at[b], gsem.at[b])

        @pl.loop(0, nchunks)
        def _(c2):
            b = lax.rem(c2, NBUF)
            pltpu.make_async_copy(
                xt_hbm.at[srcb.at[c2]], rows.at[b], gsem.at[b]).wait()
            pltpu.async_copy(rows.at[b], acc.at[dstb.at[c2]], ssem.at[b],
                             add=True)
            m = c2 - hlf

            @pl.when(m >= 0)
            def _():
                bm = lax.rem(m, NBUF)
                pltpu.make_async_copy(
                    rows.at[bm], acc.at[dstb.at[m]], ssem.at[bm]).wait()
                cm = m + NBUF

                @pl.when(cm < nchunks)
                def _():
                    pltpu.async_copy(
                        xt_hbm.at[srcb.at[cm]], rows.at[bm], gsem.at[bm])

        @pl.loop(nchunks - hlf, nchunks)
        def _(m):
            bm = lax.rem(m, NBUF)
            pltpu.make_async_copy(
                rows.at[bm], acc.at[dstb.at[m]], ssem.at[bm]).wait()

        plsc.subcore_barrier()

        obase = cid * half

        @pl.loop(0, base // zchunk)
        def _(i):
            r0 = rbase + i * zchunk
            pltpu.sync_copy(acc.at[pl.ds(r0, zchunk)], zb)
            pltpu.sync_copy(zb, sup_hbm.at[pl.ds(obase + r0, zchunk)])

        @pl.when(sid == NS - 1)
        def _():
            r0 = base * NS
            pltpu.sync_copy(acc.at[pl.ds(r0, remo)], zb.at[pl.ds(0, remo)])
            pltpu.sync_copy(zb.at[pl.ds(0, remo)],
                            sup_hbm.at[pl.ds(obase + r0, remo)])

    return k(ei4, xt)


def kernel(x, edge_index, weight, bias):
    n, d = x.shape
    e = edge_index.shape[1]
    ei4 = edge_index.reshape(2, NS, e // (NS * K), K)
    degp = _deg(ei4, n, d)
    xt = _pre(x, weight, bias.reshape(1, d))
    sup = _sup(ei4, xt)
    return _post(sup, degp)


# speedup vs baseline: 6.3071x; 1.4840x over previous
import dataclasses
import functools

import jax
import jax.numpy as jnp
from jax import lax
from jax.experimental import pallas as pl
from jax.experimental.pallas import tpu as pltpu
from jax.experimental.pallas import tpu_sc as plsc

MIN_NORM = 1e-15
BALL_EPS = 4e-3
_MAXNORM = 1.0 - BALL_EPS

BLK = 1000
NC = 2
NS = 16
K = 80
DEGW = 16
NBUF = 2


def _rownorm(x):
    return jnp.maximum(jnp.sqrt(jnp.sum(x * x, axis=-1, keepdims=True)), MIN_NORM)


def _artanh(x):
    x = jnp.clip(x, -1.0 + 1e-7, 1.0 - 1e-7)
    return 0.5 * jnp.log((1.0 + x) / (1.0 - x))


def _proj(x):
    n = _rownorm(x)
    return jnp.where(n > _MAXNORM, x / n * _MAXNORM, x)


def _expmap0(u):
    n = _rownorm(u)
    return jnp.tanh(n) * u / n


def _logmap0(p):
    n = _rownorm(p)
    return _artanh(n) / n * p


def _pre_body(x_ref, w_ref, b_ref, o_ref):
    x = x_ref[...]
    w = w_ref[...]
    b = b_ref[...]
    x_norm = _rownorm(x)
    mx = lax.dot_general(x, w, (((1,), (1,)), ((), ())),
                         preferred_element_type=jnp.float32)
    mx_norm = _rownorm(mx)
    res = jnp.tanh(mx_norm / x_norm * _artanh(x_norm)) * mx / mx_norm
    res = jnp.where(jnp.all(mx == 0.0, axis=-1, keepdims=True), 0.0, res)
    res = _proj(res)
    hb = _proj(_expmap0(b))
    x2 = jnp.sum(res * res, axis=-1, keepdims=True)
    y2 = jnp.sum(hb * hb, axis=-1, keepdims=True)
    xy = jnp.sum(res * hb, axis=-1, keepdims=True)
    num = (1.0 + 2.0 * xy + y2) * res + (1.0 - x2) * hb
    den = 1.0 + 2.0 * xy + x2 * y2
    res = _proj(num / jnp.maximum(den, MIN_NORM))
    o_ref[...] = _logmap0(res)


def _post_body(sup_ref, deg_ref, o_ref):
    s = sup_ref[...]
    deg = deg_ref[:, 0:1]
    support_t = s * (1.0 / jnp.maximum(deg, 1.0))
    h = _proj(_expmap0(support_t))
    xt = jnp.maximum(_logmap0(h), 0.0)
    o_ref[...] = _proj(_expmap0(xt))


def _pre(x, w, b2):
    n, d = x.shape
    return pl.pallas_call(
        _pre_body,
        grid=(n // BLK,),
        in_specs=[pl.BlockSpec((BLK, d), lambda i: (i, 0)),
                  pl.BlockSpec((d, d), lambda i: (0, 0)),
                  pl.BlockSpec((1, d), lambda i: (0, 0))],
        out_specs=pl.BlockSpec((BLK, d), lambda i: (i, 0)),
        out_shape=jax.ShapeDtypeStruct((n, d), jnp.float32),
    )(x, w, b2)


def _post(sup, degp):
    n, d = sup.shape
    return pl.pallas_call(
        _post_body,
        grid=(n // BLK,),
        in_specs=[pl.BlockSpec((BLK, d), lambda i: (i, 0)),
                  pl.BlockSpec((BLK, d), lambda i: (i, 0))],
        out_specs=pl.BlockSpec((BLK, d), lambda i: (i, 0)),
        out_shape=jax.ShapeDtypeStruct((n, d), jnp.float32),
    )(sup, degp)


def _localize_dst(dstb, nchunks, cid, half, garb):
    lo = cid * half

    @pl.loop(0, nchunks)
    def _(c2):
        @pl.loop(0, K // 16)
        def _(j):
            v = dstb[c2, pl.ds(j * 16, 16)] - lo
            m = (v >= 0) & (v < half)
            dstb[c2, pl.ds(j * 16, 16)] = jnp.where(m, v, garb)


def _deg(ei4, n, d):
    nchunks = ei4.shape[2]
    half = n // NC
    hrows = -(-n // d)
    hpad = 8 * (-(-hrows // 8))
    arows = hpad + 16
    base = 8 * (half // (8 * NS))
    wrows = 16 * (-(-base // 16))
    mesh = plsc.VectorSubcoreMesh(core_axis_name="c", subcore_axis_name="s")
    cp = pltpu.CompilerParams()
    if "needs_layout_passes" in pltpu.CompilerParams.__dataclass_fields__:
        cp = dataclasses.replace(cp, needs_layout_passes=False)

    @functools.partial(
        pl.kernel,
        out_type=jax.ShapeDtypeStruct((n, d), jnp.float32),
        mesh=mesh,
        compiler_params=cp,
        scratch_types=[
            pltpu.VMEM((nchunks, K), jnp.int32),
            pltpu.VMEM((hpad, d), jnp.float32),
            pltpu.VMEM((16, d), jnp.float32),
            pltpu.VMEM((wrows, d), jnp.float32),
            pltpu.VMEM((8, d), jnp.float32),
            pltpu.VMEM((8, d), jnp.float32),
            pltpu.VMEM_SHARED((arows, d), jnp.float32),
            pltpu.VMEM_SHARED((NS, hpad, d), jnp.float32),
        ],
    )
    def k(ei_hbm, deg_hbm, dstb, h, hb, wide, tb8, acc8, accd, hstage):
        cid = lax.axis_index("c")
        sid = lax.axis_index("s")
        zv = jnp.zeros((16,), jnp.float32)
        iota = lax.iota(jnp.int32, 16)

        @pl.loop(0, hpad)
        def _(i):
            @pl.loop(0, d // 16)
            def _(j):
                h[i, pl.ds(j * 16, 16)] = zv

        pltpu.sync_copy(ei_hbm.at[1, sid], dstb)

        @pl.when(sid < arows // 8)
        def _():
            pltpu.sync_copy(h.at[pl.ds(0, 8)], accd.at[pl.ds(sid * 8, 8)])

        plsc.subcore_barrier()

        ones = zv + 1.0

        @pl.loop(0, nchunks)
        def _(c2):
            @pl.loop(0, K // 16)
            def _(j):
                v = dstb[c2, pl.ds(j * 16, 16)]
                plsc.addupdate_scatter(
                    h, [jnp.right_shift(v, 7), jnp.bitwise_and(v, d - 1)],
                    ones)

        pltpu.sync_copy(h, hstage.at[sid])
        plsc.subcore_barrier()

        @pl.when(sid < hpad // 8)
        def _():
            r0 = sid * 8

            @pl.loop(0, 8)
            def _(i):
                @pl.loop(0, d // 16)
                def _(j):
                    acc8[i, pl.ds(j * 16, 16)] = zv

            @pl.loop(0, NS)
            def _(u):
                pltpu.sync_copy(hstage.at[u, pl.ds(r0, 8)], tb8)

                @pl.loop(0, 8)
                def _(i):
                    @pl.loop(0, d // 16)
                    def _(j):
                        acc8[i, pl.ds(j * 16, 16)] += tb8[i,
                                                          pl.ds(j * 16, 16)]

            pltpu.sync_copy(acc8, accd.at[pl.ds(r0, 8)])

        plsc.subcore_barrier()

        p0 = cid * half + sid * base
        r0a = jnp.bitwise_and(jnp.right_shift(p0, 7), -8)
        pltpu.sync_copy(accd.at[pl.ds(r0a, 16)], hb)
        q0 = p0 - r0a * d

        @pl.loop(0, wrows // 16)
        def _(kk):
            q = iota + (q0 + kk * 16)
            dv = plsc.load_gather(
                hb, [jnp.right_shift(q, 7), jnp.bitwise_and(q, d - 1)])
            plsc.store_scatter(wide, [iota + kk * 16,
                                      jnp.zeros((16,), jnp.int32)], dv)

        pltpu.sync_copy(wide.at[pl.ds(0, base)], deg_hbm.at[pl.ds(p0, base)])

        @pl.when(sid == NS - 1)
        def _():
            pltpu.sync_copy(wide.at[pl.ds(base, wrows - base)],
                            deg_hbm.at[pl.ds(p0 + base, wrows - base)])

    return k(ei4)


def _sup(ei4, xt):
    n, d = xt.shape
    nchunks = ei4.shape[2]
    half = n // NC
    nacc = half + 8
    garb = half
    base = 8 * (half // (8 * NS))
    remz = nacc - base * NS
    remo = half - base * NS
    zchunk = 24
    mesh = plsc.VectorSubcoreMesh(core_axis_name="c", subcore_axis_name="s")
    cp = pltpu.CompilerParams()
    if "needs_layout_passes" in pltpu.CompilerParams.__dataclass_fields__:
        cp = dataclasses.replace(cp, needs_layout_passes=False)

    @functools.partial(
        pl.kernel,
        out_type=jax.ShapeDtypeStruct((n, d), jnp.float32),
        mesh=mesh,
        compiler_params=cp,
        scratch_types=[
            pltpu.VMEM((nchunks, K), jnp.int32),
            pltpu.VMEM((nchunks, K), jnp.int32),
            pltpu.VMEM((NBUF, K, d), jnp.float32),
            pltpu.VMEM((zchunk, d), jnp.float32),
            pltpu.VMEM_SHARED((nacc, d), jnp.float32),
            pltpu.SemaphoreType.DMA((NBUF,)),
            pltpu.SemaphoreType.DMA((NBUF,)),
        ],
    )
    def k(ei_hbm, xt_hbm, sup_hbm, srcb, dstb, rows, zb, acc, gsem, ssem):
        cid = lax.axis_index("c")
        sid = lax.axis_index("s")
        zv = jnp.zeros((16,), jnp.float32)

        @pl.loop(0, zchunk)
        def _(i):
            @pl.loop(0, d // 16)
            def _(j):
                zb[i, pl.ds(j * 16, 16)] = zv

        rbase = sid * base

        @pl.loop(0, base // zchunk)
        def _(i):
            pltpu.sync_copy(zb, acc.at[pl.ds(rbase + i * zchunk, zchunk)])

        @pl.when(sid == NS - 1)
        def _():
            pltpu.sync_copy(zb.at[pl.ds(0, remz)],
                            acc.at[pl.ds(nacc - remz, remz)])

        pltpu.sync_copy(ei_hbm.at[0, sid], srcb)
        pltpu.sync_copy(ei_hbm.at[1, sid], dstb)
        _localize_dst(dstb, nchunks, cid, half, garb)
        plsc.subcore_barrier()

        hlf = NBUF // 2
        for b in range(NBUF):
            pltpu.async_copy(xt_hbm.at[srcb.at[b]], rows.at[b], gsem.at[b])

        @pl.loop(0, nchunks)
        def _(c2):
            b = lax.rem(c2, NBUF)
            pltpu.make_async_copy(
                xt_hbm.at[srcb.at[c2]], rows.at[b], gsem.at[b]).wait()
            pltpu.async_copy(rows.at[b], acc.at[dstb.at[c2]], ssem.at[b],
                             add=True)
            m = c2 - hlf

            @pl.when(m >= 0)
            def _():
                bm = lax.rem(m, NBUF)
                pltpu.make_async_copy(
                    rows.at[bm], acc.at[dstb.at[m]], ssem.at[bm]).wait()
                cm = m + NBUF

                @pl.when(cm < nchunks)
                def _():
                    pltpu.async_copy(
                        xt_hbm.at[srcb.at[cm]], rows.at[bm], gsem.at[bm])

        @pl.loop(nchunks - hlf, nchunks)
        def _(m):
            bm = lax.rem(m, NBUF)
            pltpu.make_async_copy(
                rows.at[bm], acc.at[dstb.at[m]], ssem.at[bm]).wait()

        plsc.subcore_barrier()

        obase = cid * half

        @pl.loop(0, base // zchunk)
        def _(i):
            r0 = rbase + i * zchunk
            pltpu.sync_copy(acc.at[pl.ds(r0, zchunk)], zb)
            pltpu.sync_copy(zb, sup_hbm.at[pl.ds(obase + r0, zchunk)])

        @pl.when(sid == NS - 1)
        def _():
            r0 = base * NS
            pltpu.sync_copy(acc.at[pl.ds(r0, remo)], zb.at[pl.ds(0, remo)])
            pltpu.sync_copy(zb.at[pl.ds(0, remo)],
                            sup_hbm.at[pl.ds(obase + r0, remo)])

    return k(ei4, xt)


def kernel(x, edge_index, weight, bias):
    n, d = x.shape
    e = edge_index.shape[1]
    ei4 = edge_index.reshape(2, NS, e // (NS * K), K)
    degp = _deg(ei4, n, d)
    xt = _pre(x, weight, bias.reshape(1, d))
    sup = _sup(ei4, xt)
    return _post(sup, degp)
